# baseline (speedup 1.0000x reference)
import jax
import jax.numpy as jnp
from jax import lax
from jax.experimental import pallas as pl
from jax.experimental.pallas import tpu as pltpu
from jax.experimental.pallas import tpu_sc as plsc

D = 1024
B = 16
L = 1024
N_ROWS = B * L
NW = 32
W_ROWS = N_ROWS // NW
CH = 16
NCH = W_ROWS // CH
NV = D // 16
LB = 3
E_ROWS = 2 * L
EW = E_ROWS // NW
RB = 256
NG = D // 32
SCALE = 0.03125
_ILV = plsc.PackFormat.INTERLEAVED

_SC_PARAMS = pltpu.CompilerParams(needs_layout_passes=False)
_MESH = dict(core_axis_name="c", subcore_axis_name="s")


def _wid():
    return lax.axis_index("s") * 2 + lax.axis_index("c")


def _enc0_body(pos0_hbm, pe_hbm, enc0_out, idx_buf, gat_buf, row_buf, sem):
    base = _wid() * EW
    pltpu.sync_copy(pos0_hbm.at[pl.ds(base, EW)], idx_buf)
    pltpu.async_copy(pe_hbm.at[idx_buf], gat_buf, sem).wait()

    def row(r, carry):
        for g in range(NG):
            w = plsc.bitcast(gat_buf[r, pl.ds(g * 16, 16)], jnp.bfloat16)
            lo, hi = plsc.unpack(w, format=_ILV)
            row_buf[r, pl.ds(g * 32, 16)] = lo * SCALE
            row_buf[r, pl.ds(g * 32 + 16, 16)] = hi * SCALE
        return carry

    lax.fori_loop(0, EW, row, 0)
    pltpu.sync_copy(row_buf, enc0_out.at[pl.ds(base, EW)])


def _build_enc0(pos0, pe_i32):
    return pl.kernel(
        _enc0_body,
        jax.ShapeDtypeStruct((E_ROWS, D), jnp.float32),
        mesh=plsc.VectorSubcoreMesh(**_MESH),
        compiler_params=_SC_PARAMS,
        scratch_types=[
            pltpu.VMEM((EW,), jnp.int32),
            pltpu.VMEM((EW, D // 2), jnp.int32),
            pltpu.VMEM((EW, D), jnp.float32),
            pltpu.SemaphoreType.DMA,
        ],
    )(pos0, pe_i32)


def _lang_body(lang_hbm, pos_lang_hbm, pe_hbm, lang_out,
               idx_all, in_buf, pe_buf,
               sem_pe0, sem_pe1, sem_pe2, sem_ld0, sem_ld1, sem_ld2,
               sem_st0, sem_st1, sem_st2):
    base = _wid() * W_ROWS
    sem_pe = [sem_pe0, sem_pe1, sem_pe2]
    sem_ld = [sem_ld0, sem_ld1, sem_ld2]
    sem_st = [sem_st0, sem_st1, sem_st2]

    pltpu.sync_copy(pos_lang_hbm.at[pl.ds(base, W_ROWS)], idx_all)

    def idx_ref(k):
        return idx_all.at[pl.ds(k * CH, CH)]

    def start(k, p):
        row0 = base + k * CH
        pltpu.async_copy(pe_hbm.at[idx_ref(k)], pe_buf.at[p], sem_pe[p])
        pltpu.async_copy(lang_hbm.at[pl.ds(row0, CH)], in_buf.at[p], sem_ld[p])

    def wait_loads(k, p):
        row0 = base + k * CH
        pltpu.make_async_copy(pe_hbm.at[idx_ref(k)], pe_buf.at[p], sem_pe[p]).wait()
        pltpu.make_async_copy(lang_hbm.at[pl.ds(row0, CH)], in_buf.at[p], sem_ld[p]).wait()

    def compute(p):
        def row(r, carry):
            for g in range(NG):
                w = plsc.bitcast(pe_buf[p, r, pl.ds(g * 16, 16)], jnp.bfloat16)
                lo, hi = plsc.unpack(w, format=_ILV)
                plsc.addupdate(in_buf.at[p, r, pl.ds(g * 32, 16)], lo * SCALE)
                plsc.addupdate(in_buf.at[p, r, pl.ds(g * 32 + 16, 16)], hi * SCALE)
            return carry

        lax.fori_loop(0, CH, row, 0)

    def start_store(k, p):
        row0 = base + k * CH
        pltpu.async_copy(in_buf.at[p], lang_out.at[pl.ds(row0, CH)], sem_st[p])

    def wait_store(k, p):
        row0 = base + k * CH
        pltpu.make_async_copy(in_buf.at[p], lang_out.at[pl.ds(row0, CH)], sem_st[p]).wait()

    start(0, 0)

    def step(k, p):
        knext = k + 1
        pnext = (p + 1) % LB

        @pl.when(knext < NCH)
        def _():
            @pl.when(knext >= LB)
            def _():
                wait_store(knext - LB, pnext)
            start(knext, pnext)

        wait_loads(k, p)
        compute(p)
        start_store(k, p)

    def outer(i, carry):
        kk = i * LB
        for p in range(LB):
            step(kk + p, p)
        return carry

    nfull = NCH - NCH % LB
    lax.fori_loop(0, nfull // LB, outer, 0)
    for k in range(nfull, NCH):
        step(k, k % LB)
    for k in range(NCH - LB, NCH):
        wait_store(k, k % LB)


def _run_lang(lang2d, pos_lang, pe):
    return pl.kernel(
        _lang_body,
        jax.ShapeDtypeStruct((N_ROWS, D), jnp.float32),
        mesh=plsc.VectorSubcoreMesh(**_MESH),
        compiler_params=_SC_PARAMS,
        scratch_types=[
            pltpu.VMEM((W_ROWS,), jnp.int32),
            pltpu.VMEM((LB, CH, D), jnp.float32),
            pltpu.VMEM((LB, CH, D // 2), jnp.int32),
        ] + [pltpu.SemaphoreType.DMA] * 9,
    )(lang2d, pos_lang, pe)


def _fa_body(lens_ref, frames_ref, actions_ref, enc0_ref, frames_o, actions_o):
    b = pl.program_id(0)
    j = pl.program_id(1)
    lb = lens_ref[b]
    rem = lax.rem(lb, 8)
    al = pl.multiple_of(lb - rem + j * RB, 8)
    enc_wide = enc0_ref[pl.ds(al, RB + 8), :]
    enc = pltpu.roll(enc_wide, (RB + 8) - rem, axis=0)[:RB]
    frames_o[...] = frames_ref[...] + enc
    actions_o[...] = actions_ref[...] + enc


def _run_fa(frames2d, actions2d, enc0, lens_i32):
    blk = pl.BlockSpec((RB, D), lambda b, j, lens: (b * (L // RB) + j, 0))
    grid_spec = pltpu.PrefetchScalarGridSpec(
        num_scalar_prefetch=1,
        grid=(B, L // RB),
        in_specs=[
            blk,
            blk,
            pl.BlockSpec((E_ROWS, D), lambda b, j, lens: (0, 0)),
        ],
        out_specs=[blk, blk],
    )
    out_sds = jax.ShapeDtypeStruct((N_ROWS, D), jnp.float32)
    return pl.pallas_call(
        _fa_body,
        grid_spec=grid_spec,
        out_shape=[out_sds, out_sds],
    )(lens_i32, frames2d, actions2d, enc0)


def kernel(lang, frames, actions, lens_lang, lens_frames, pos, pe):
    del lens_frames
    pos = pos.astype(jnp.int32)
    lens_i32 = lens_lang.astype(jnp.int32)
    lang2d = lang.reshape(N_ROWS, D)
    pos_lang = pos[:, :L].reshape(N_ROWS)
    pos0 = pos[0]
    pe_i32 = lax.bitcast_convert_type(
        pe.astype(jnp.bfloat16)
        .reshape(-1, NG, 2, 16).transpose(0, 1, 3, 2).reshape(-1, D // 2, 2),
        jnp.int32)

    enc0 = _build_enc0(pos0, pe_i32)
    lang_out = _run_lang(lang2d, pos_lang, pe_i32)
    fo, ao = _run_fa(frames.reshape(N_ROWS, D),
                     actions.reshape(N_ROWS, D),
                     enc0, lens_i32)
    return (lang_out.reshape(B, L, D),
            fo.reshape(B, L, D),
            ao.reshape(B, L, D))


# speedup vs baseline: 1.2083x; 1.2083x over previous
import jax
import jax.numpy as jnp
from jax import lax
from jax.experimental import pallas as pl
from jax.experimental.pallas import tpu as pltpu
from jax.experimental.pallas import tpu_sc as plsc

D = 1024
B = 16
L = 1024
N_ROWS = B * L
NW = 32
W_ROWS = N_ROWS // NW
CH = 8
NCH = W_ROWS // CH
NV = D // 16
LB = 6
PF = LB - 2
E_ROWS = 2 * L
EW = E_ROWS // NW
RB = 256
NG = D // 32
SCALE = 0.03125
_ILV = plsc.PackFormat.INTERLEAVED

_SC_PARAMS = pltpu.CompilerParams(needs_layout_passes=False)
_MESH = dict(core_axis_name="c", subcore_axis_name="s")


def _wid():
    return lax.axis_index("s") * 2 + lax.axis_index("c")


def _enc0_body(pos0_hbm, pe_hbm, enc0_out, idx_buf, row_buf, sem):
    base = _wid() * EW
    pltpu.sync_copy(pos0_hbm.at[pl.ds(base, EW)], idx_buf)
    pltpu.async_copy(pe_hbm.at[idx_buf], row_buf, sem).wait()

    def row(r, carry):
        for d in range(NV):
            sl = pl.ds(d * 16, 16)
            row_buf[r, sl] = row_buf[r, sl] * SCALE
        return carry

    lax.fori_loop(0, EW, row, 0)
    pltpu.sync_copy(row_buf, enc0_out.at[pl.ds(base, EW)])


def _build_enc0(pos0, pe):
    return pl.kernel(
        _enc0_body,
        jax.ShapeDtypeStruct((E_ROWS, D), jnp.float32),
        mesh=plsc.VectorSubcoreMesh(**_MESH),
        compiler_params=_SC_PARAMS,
        scratch_types=[
            pltpu.VMEM((EW,), jnp.int32),
            pltpu.VMEM((EW, D), jnp.float32),
            pltpu.SemaphoreType.DMA,
        ],
    )(pos0, pe)


def _lang_body(lang_hbm, pos_lang_hbm, pe_hbm, lang_out,
               idx_all, in_buf, pe_buf, *sems):
    base = _wid() * W_ROWS
    sem_pe = sems[:LB]
    sem_ld = sems[LB:2 * LB]
    sem_st = sems[2 * LB:]

    pltpu.sync_copy(pos_lang_hbm.at[pl.ds(base, W_ROWS)], idx_all)

    def idx_ref(k):
        return idx_all.at[pl.ds(k * CH, CH)]

    def start(k, p):
        row0 = base + k * CH
        pltpu.async_copy(pe_hbm.at[idx_ref(k)], pe_buf.at[p], sem_pe[p])
        pltpu.async_copy(lang_hbm.at[pl.ds(row0, CH)], in_buf.at[p], sem_ld[p])

    def wait_loads(k, p):
        row0 = base + k * CH
        pltpu.make_async_copy(pe_hbm.at[idx_ref(k)], pe_buf.at[p], sem_pe[p]).wait()
        pltpu.make_async_copy(lang_hbm.at[pl.ds(row0, CH)], in_buf.at[p], sem_ld[p]).wait()

    def compute(p):
        def row(r, carry):
            for d in range(NV):
                sl = pl.ds(d * 16, 16)
                plsc.addupdate(in_buf.at[p, r, sl], pe_buf[p, r, sl] * SCALE)
            return carry

        lax.fori_loop(0, CH, row, 0)

    def start_store(k, p):
        row0 = base + k * CH
        pltpu.async_copy(in_buf.at[p], lang_out.at[pl.ds(row0, CH)], sem_st[p])

    def wait_store(k, p):
        row0 = base + k * CH
        pltpu.make_async_copy(in_buf.at[p], lang_out.at[pl.ds(row0, CH)], sem_st[p]).wait()

    for d in range(PF):
        start(d, d % LB)

    def step(k, p):
        kpre = k + PF
        ppre = (p + PF) % LB

        @pl.when(kpre < NCH)
        def _():
            @pl.when(kpre >= LB)
            def _():
                wait_store(kpre - LB, ppre)
            start(kpre, ppre)

        wait_loads(k, p)
        compute(p)
        start_store(k, p)

    def outer(i, carry):
        kk = i * LB
        for p in range(LB):
            step(kk + p, p)
        return carry

    nfull = NCH - NCH % LB
    lax.fori_loop(0, nfull // LB, outer, 0)
    for k in range(nfull, NCH):
        step(k, k % LB)
    for k in range(NCH - LB, NCH):
        wait_store(k, k % LB)


def _run_lang(lang2d, pos_lang, pe):
    return pl.kernel(
        _lang_body,
        jax.ShapeDtypeStruct((N_ROWS, D), jnp.float32),
        mesh=plsc.VectorSubcoreMesh(**_MESH),
        compiler_params=_SC_PARAMS,
        scratch_types=[
            pltpu.VMEM((W_ROWS,), jnp.int32),
            pltpu.VMEM((LB, CH, D), jnp.float32),
            pltpu.VMEM((LB, CH, D), jnp.float32),
        ] + [pltpu.SemaphoreType.DMA] * (3 * LB),
    )(lang2d, pos_lang, pe)


def _fa_body(lens_ref, frames_ref, actions_ref, enc0_ref, frames_o, actions_o):
    b = pl.program_id(0)
    j = pl.program_id(1)
    lb = lens_ref[b]
    rem = lax.rem(lb, 8)
    al = pl.multiple_of(lb - rem + j * RB, 8)
    enc_wide = enc0_ref[pl.ds(al, RB + 8), :]
    enc = pltpu.roll(enc_wide, (RB + 8) - rem, axis=0)[:RB]
    frames_o[...] = frames_ref[...] + enc
    actions_o[...] = actions_ref[...] + enc


def _run_fa(frames2d, actions2d, enc0, lens_i32):
    blk = pl.BlockSpec((RB, D), lambda b, j, lens: (b * (L // RB) + j, 0))
    grid_spec = pltpu.PrefetchScalarGridSpec(
        num_scalar_prefetch=1,
        grid=(B, L // RB),
        in_specs=[
            blk,
            blk,
            pl.BlockSpec((E_ROWS, D), lambda b, j, lens: (0, 0)),
        ],
        out_specs=[blk, blk],
    )
    out_sds = jax.ShapeDtypeStruct((N_ROWS, D), jnp.float32)
    return pl.pallas_call(
        _fa_body,
        grid_spec=grid_spec,
        out_shape=[out_sds, out_sds],
    )(lens_i32, frames2d, actions2d, enc0)


def kernel(lang, frames, actions, lens_lang, lens_frames, pos, pe):
    del lens_frames
    pos = pos.astype(jnp.int32)
    lens_i32 = lens_lang.astype(jnp.int32)
    lang2d = lang.reshape(N_ROWS, D)
    pos_lang = pos[:, :L].reshape(N_ROWS)
    pos0 = pos[0]
    enc0 = _build_enc0(pos0, pe)
    lang_out = _run_lang(lang2d, pos_lang, pe)
    fo, ao = _run_fa(frames.reshape(N_ROWS, D),
                     actions.reshape(N_ROWS, D),
                     enc0, lens_i32)
    return (lang_out.reshape(B, L, D),
            fo.reshape(B, L, D),
            ao.reshape(B, L, D))


# speedup vs baseline: 1.2212x; 1.0107x over previous
import jax
import jax.numpy as jnp
from jax import lax
from jax.experimental import pallas as pl
from jax.experimental.pallas import tpu as pltpu
from jax.experimental.pallas import tpu_sc as plsc

D = 1024
B = 16
L = 1024
N_ROWS = B * L
NW = 32
W_ROWS = N_ROWS // NW
CH = 16
NCH = W_ROWS // CH
NV = D // 16
LB = 3
PF = 1
E_ROWS = 2 * L
EW = E_ROWS // NW
RB = 256
NG = D // 32
SCALE = 0.03125
_ILV = plsc.PackFormat.INTERLEAVED

_SC_PARAMS = pltpu.CompilerParams(needs_layout_passes=False)
_MESH = dict(core_axis_name="c", subcore_axis_name="s")


def _wid():
    return lax.axis_index("s") * 2 + lax.axis_index("c")


def _enc0_body(pos0_hbm, pe_hbm, enc0_out, idx_buf, row_buf, sem):
    base = _wid() * EW
    pltpu.sync_copy(pos0_hbm.at[pl.ds(base, EW)], idx_buf)
    pltpu.async_copy(pe_hbm.at[idx_buf], row_buf, sem).wait()

    def row(r, carry):
        for d in range(NV):
            sl = pl.ds(d * 16, 16)
            row_buf[r, sl] = row_buf[r, sl] * SCALE
        return carry

    lax.fori_loop(0, EW, row, 0)
    pltpu.sync_copy(row_buf, enc0_out.at[pl.ds(base, EW)])


def _build_enc0(pos0, pe):
    return pl.kernel(
        _enc0_body,
        jax.ShapeDtypeStruct((E_ROWS, D), jnp.float32),
        mesh=plsc.VectorSubcoreMesh(**_MESH),
        compiler_params=_SC_PARAMS,
        scratch_types=[
            pltpu.VMEM((EW,), jnp.int32),
            pltpu.VMEM((EW, D), jnp.float32),
            pltpu.SemaphoreType.DMA,
        ],
    )(pos0, pe)


def _lang_body(lang_hbm, pos_lang_hbm, pe_hbm, lang_out,
               idx_all, in_buf, pe_buf, *sems):
    base = _wid() * W_ROWS
    sem_pe = sems[:LB]
    sem_ld = sems[LB:2 * LB]
    sem_st = sems[2 * LB:]

    pltpu.sync_copy(pos_lang_hbm.at[pl.ds(base, W_ROWS)], idx_all)

    def idx_ref(k):
        return idx_all.at[pl.ds(k * CH, CH)]

    def start(k, p):
        row0 = base + k * CH
        pltpu.async_copy(pe_hbm.at[idx_ref(k)], pe_buf.at[p], sem_pe[p])
        pltpu.async_copy(lang_hbm.at[pl.ds(row0, CH)], in_buf.at[p], sem_ld[p])

    def wait_loads(k, p):
        row0 = base + k * CH
        pltpu.make_async_copy(pe_hbm.at[idx_ref(k)], pe_buf.at[p], sem_pe[p]).wait()
        pltpu.make_async_copy(lang_hbm.at[pl.ds(row0, CH)], in_buf.at[p], sem_ld[p]).wait()

    def compute(p):
        def row(r, carry):
            for d in range(NV):
                sl = pl.ds(d * 16, 16)
                plsc.addupdate(in_buf.at[p, r, sl], pe_buf[p, r, sl] * SCALE)
            return carry

        lax.fori_loop(0, CH, row, 0)

    def start_store(k, p):
        row0 = base + k * CH
        pltpu.async_copy(in_buf.at[p], lang_out.at[pl.ds(row0, CH)], sem_st[p])

    def wait_store(k, p):
        row0 = base + k * CH
        pltpu.make_async_copy(in_buf.at[p], lang_out.at[pl.ds(row0, CH)], sem_st[p]).wait()

    for d in range(PF):
        start(d, d % LB)

    def step(k, p):
        kpre = k + PF
        ppre = (p + PF) % LB

        @pl.when(kpre < NCH)
        def _():
            @pl.when(kpre >= LB)
            def _():
                wait_store(kpre - LB, ppre)
            start(kpre, ppre)

        wait_loads(k, p)
        compute(p)
        start_store(k, p)

    def outer(i, carry):
        kk = i * LB
        for p in range(LB):
            step(kk + p, p)
        return carry

    nfull = NCH - NCH % LB
    lax.fori_loop(0, nfull // LB, outer, 0)
    for k in range(nfull, NCH):
        step(k, k % LB)
    for k in range(NCH - LB, NCH):
        wait_store(k, k % LB)


def _run_lang(lang2d, pos_lang, pe):
    return pl.kernel(
        _lang_body,
        jax.ShapeDtypeStruct((N_ROWS, D), jnp.float32),
        mesh=plsc.VectorSubcoreMesh(**_MESH),
        compiler_params=_SC_PARAMS,
        scratch_types=[
            pltpu.VMEM((W_ROWS,), jnp.int32),
            pltpu.VMEM((LB, CH, D), jnp.float32),
            pltpu.VMEM((LB, CH, D), jnp.float32),
        ] + [pltpu.SemaphoreType.DMA] * (3 * LB),
    )(lang2d, pos_lang, pe)


def _fa_body(lens_ref, frames_ref, actions_ref, enc0_ref, frames_o, actions_o):
    b = pl.program_id(0)
    j = pl.program_id(1)
    lb = lens_ref[b]
    rem = lax.rem(lb, 8)
    al = pl.multiple_of(lb - rem + j * RB, 8)
    for t in range(RB // 8):
        w = enc0_ref[pl.ds(al + t * 8, 16), :]
        enc8 = pltpu.roll(w, 16 - rem, axis=0)[:8]
        sl = pl.ds(t * 8, 8)
        frames_o[sl, :] = frames_ref[sl, :] + enc8
        actions_o[sl, :] = actions_ref[sl, :] + enc8


def _run_fa(frames2d, actions2d, enc0, lens_i32):
    blk = pl.BlockSpec((RB, D), lambda b, j, lens: (b * (L // RB) + j, 0))
    grid_spec = pltpu.PrefetchScalarGridSpec(
        num_scalar_prefetch=1,
        grid=(B, L // RB),
        in_specs=[
            blk,
            blk,
            pl.BlockSpec((E_ROWS, D), lambda b, j, lens: (0, 0)),
        ],
        out_specs=[blk, blk],
    )
    out_sds = jax.ShapeDtypeStruct((N_ROWS, D), jnp.float32)
    return pl.pallas_call(
        _fa_body,
        grid_spec=grid_spec,
        out_shape=[out_sds, out_sds],
    )(lens_i32, frames2d, actions2d, enc0)


def kernel(lang, frames, actions, lens_lang, lens_frames, pos, pe):
    del lens_frames
    pos = pos.astype(jnp.int32)
    lens_i32 = lens_lang.astype(jnp.int32)
    lang2d = lang.reshape(N_ROWS, D)
    pos_lang = pos[:, :L].reshape(N_ROWS)
    pos0 = pos[0]
    enc0 = _build_enc0(pos0, pe)
    lang_out = _run_lang(lang2d, pos_lang, pe)
    fo, ao = _run_fa(frames.reshape(N_ROWS, D),
                     actions.reshape(N_ROWS, D),
                     enc0, lens_i32)
    return (lang_out.reshape(B, L, D),
            fo.reshape(B, L, D),
            ao.reshape(B, L, D))
